# baseline (speedup 1.0000x reference)
import functools

import jax
import jax.numpy as jnp
from jax import lax
from jax.experimental import pallas as pl
from jax.experimental.pallas import tpu as pltpu
from jax.experimental.pallas import tpu_sc as plsc

H = 128
NF = 50000
NC = 5000
E = 500000

NFP = 50176
NCP = 5120
EP = 524288
EB = 128
SEG = 32
CH = 12544
L = 16

_GATE_R = EP // H


def _gate_body(ea_fc_ref, ea_cf_ref, wm_ref, bm_ref, wfc_ref, wcf_ref):
    wm = wm_ref[0, 0]
    bm = bm_ref[0, 0]
    r = lax.broadcasted_iota(jnp.int32, (_GATE_R, H), 0)
    c = lax.broadcasted_iota(jnp.int32, (_GATE_R, H), 1)
    valid = (r * H + c) < E
    for ea_ref, w_ref in ((ea_fc_ref, wfc_ref), (ea_cf_ref, wcf_ref)):
        w = jax.nn.sigmoid(ea_ref[...] * wm + bm)
        w_ref[...] = jnp.where(valid, w, 0.0)


def _gates(ea_fc, ea_cf, W_mix, b_mix):
    return pl.pallas_call(
        _gate_body,
        out_shape=(
            jax.ShapeDtypeStruct((_GATE_R, H), jnp.float32),
            jax.ShapeDtypeStruct((_GATE_R, H), jnp.float32),
        ),
        in_specs=[
            pl.BlockSpec((_GATE_R, H), lambda: (0, 0)),
            pl.BlockSpec((_GATE_R, H), lambda: (0, 0)),
            pl.BlockSpec(memory_space=pltpu.SMEM),
            pl.BlockSpec(memory_space=pltpu.SMEM),
        ],
        out_specs=(
            pl.BlockSpec((_GATE_R, H), lambda: (0, 0)),
            pl.BlockSpec((_GATE_R, H), lambda: (0, 0)),
        ),
    )(ea_fc, ea_cf, W_mix, b_mix.reshape(1, 1))


def _ln_rows(x, g, b):
    m = jnp.mean(x, axis=-1, keepdims=True)
    v = jnp.mean(jnp.square(x - m), axis=-1, keepdims=True)
    return g * (x - m) / jnp.sqrt(v + 1e-5) + b


def _enc_body(x_ref, W_ref, b_ref, g_ref, be_ref, o_ref):
    x = jnp.dot(x_ref[...], W_ref[...].T, preferred_element_type=jnp.float32)
    x = jax.nn.relu(x + b_ref[...])
    o_ref[...] = _ln_rows(x, g_ref[...], be_ref[...])


def _encode(x, W, b, g, be, rows, blk=512):
    return pl.pallas_call(
        _enc_body,
        grid=(rows // blk,),
        out_shape=jax.ShapeDtypeStruct((rows, H), jnp.float32),
        in_specs=[
            pl.BlockSpec((blk, H), lambda i: (i, 0)),
            pl.BlockSpec((H, H), lambda i: (0, 0)),
            pl.BlockSpec((1, H), lambda i: (0, 0)),
            pl.BlockSpec((1, H), lambda i: (0, 0)),
            pl.BlockSpec((1, H), lambda i: (0, 0)),
        ],
        out_specs=pl.BlockSpec((blk, H), lambda i: (i, 0)),
    )(x, W, b.reshape(1, H), g.reshape(1, H), be.reshape(1, H))


def _upd_f_body(agg_ref, h_ref, Wrel_ref, brel_ref, Wroot_ref, g_ref, b_ref,
                o_ref, sum_ref, *, nvalid, blk):
    i = pl.program_id(0)
    x = jnp.dot(agg_ref[...], Wrel_ref[...].T, preferred_element_type=jnp.float32)
    x = x + brel_ref[...]
    x = x + jnp.dot(h_ref[...], Wroot_ref[...].T, preferred_element_type=jnp.float32)
    x = jax.nn.relu(x)
    out = _ln_rows(x, g_ref[...], b_ref[...])
    o_ref[...] = out
    rowid = i * blk + lax.broadcasted_iota(jnp.int32, (blk, 1), 0)
    masked = jnp.where(rowid < nvalid, out, 0.0)

    @pl.when(i == 0)
    def _():
        sum_ref[...] = jnp.zeros_like(sum_ref)

    sum_ref[...] += jnp.sum(masked, axis=0, keepdims=True)


def _upd_c_body(aggp_ref, h_ref, Wrel_ref, brel_ref, Wroot_ref, g_ref, b_ref,
                o_ref, sum_ref, *, nvalid, blk):
    i = pl.program_id(0)
    agg = aggp_ref[0] + aggp_ref[1]
    x = jnp.dot(agg, Wrel_ref[...].T, preferred_element_type=jnp.float32)
    x = x + brel_ref[...]
    x = x + jnp.dot(h_ref[...], Wroot_ref[...].T, preferred_element_type=jnp.float32)
    x = jax.nn.relu(x)
    out = _ln_rows(x, g_ref[...], b_ref[...])
    o_ref[...] = out
    rowid = i * blk + lax.broadcasted_iota(jnp.int32, (blk, 1), 0)
    masked = jnp.where(rowid < nvalid, out, 0.0)

    @pl.when(i == 0)
    def _():
        sum_ref[...] = jnp.zeros_like(sum_ref)

    sum_ref[...] += jnp.sum(masked, axis=0, keepdims=True)


def _update_f(agg, h, Wrel, brel, Wroot, g, b, blk=512):
    return pl.pallas_call(
        functools.partial(_upd_f_body, nvalid=NF, blk=blk),
        grid=(NFP // blk,),
        out_shape=(
            jax.ShapeDtypeStruct((NFP, H), jnp.float32),
            jax.ShapeDtypeStruct((1, H), jnp.float32),
        ),
        in_specs=[
            pl.BlockSpec((blk, H), lambda i: (i, 0)),
            pl.BlockSpec((blk, H), lambda i: (i, 0)),
            pl.BlockSpec((H, H), lambda i: (0, 0)),
            pl.BlockSpec((1, H), lambda i: (0, 0)),
            pl.BlockSpec((H, H), lambda i: (0, 0)),
            pl.BlockSpec((1, H), lambda i: (0, 0)),
            pl.BlockSpec((1, H), lambda i: (0, 0)),
        ],
        out_specs=(
            pl.BlockSpec((blk, H), lambda i: (i, 0)),
            pl.BlockSpec((1, H), lambda i: (0, 0)),
        ),
    )(agg, h, Wrel, brel.reshape(1, H), Wroot, g.reshape(1, H),
      b.reshape(1, H))


def _update_c(aggp, h, Wrel, brel, Wroot, g, b, blk=512):
    return pl.pallas_call(
        functools.partial(_upd_c_body, nvalid=NC, blk=blk),
        grid=(NCP // blk,),
        out_shape=(
            jax.ShapeDtypeStruct((NCP, H), jnp.float32),
            jax.ShapeDtypeStruct((1, H), jnp.float32),
        ),
        in_specs=[
            pl.BlockSpec((2, blk, H), lambda i: (0, i, 0)),
            pl.BlockSpec((blk, H), lambda i: (i, 0)),
            pl.BlockSpec((H, H), lambda i: (0, 0)),
            pl.BlockSpec((1, H), lambda i: (0, 0)),
            pl.BlockSpec((H, H), lambda i: (0, 0)),
            pl.BlockSpec((1, H), lambda i: (0, 0)),
            pl.BlockSpec((1, H), lambda i: (0, 0)),
        ],
        out_specs=(
            pl.BlockSpec((blk, H), lambda i: (i, 0)),
            pl.BlockSpec((1, H), lambda i: (0, 0)),
        ),
    )(aggp, h, Wrel, brel.reshape(1, H), Wroot, g.reshape(1, H),
      b.reshape(1, H))


def _readout_body(sf_ref, sc_ref, Wg_ref, bg_ref, Wc_ref, bc_ref, o_ref):
    fp = sf_ref[...] / NF
    cp = sc_ref[...] / NC
    z = (jnp.sum(fp * Wg_ref[0:1, 0:H]) + jnp.sum(cp * Wg_ref[0:1, H:2 * H])
         + bg_ref[0, 0])
    alpha = jax.nn.sigmoid(z)
    pooled = alpha * fp + (1.0 - alpha) * cp
    o_ref[...] = (jnp.sum(pooled * Wc_ref[...])
                  + bc_ref[0, 0]).reshape(1, 1)


def _readout(sum_f, sum_c, W_gate, b_gate, W_cls, b_cls):
    return pl.pallas_call(
        _readout_body,
        out_shape=jax.ShapeDtypeStruct((1, 1), jnp.float32),
        in_specs=[
            pl.BlockSpec((1, H), lambda: (0, 0)),
            pl.BlockSpec((1, H), lambda: (0, 0)),
            pl.BlockSpec((1, 2 * H), lambda: (0, 0)),
            pl.BlockSpec(memory_space=pltpu.SMEM),
            pl.BlockSpec((1, H), lambda: (0, 0)),
            pl.BlockSpec(memory_space=pltpu.SMEM),
        ],
        out_specs=pl.BlockSpec((1, 1), lambda: (0, 0)),
    )(sum_f, sum_c, W_gate, b_gate.reshape(1, 1), W_cls, b_cls.reshape(1, 1))


def _zero_rows(rows_v, n):
    def zrow(e, _):
        for j in range(H // L):
            rows_v[e, pl.ds(j * L, L)] = jnp.zeros((L,), jnp.float32)
        return 0
    lax.fori_loop(0, n, zrow, 0)


def _drain(hbm_ref, vmem_ref, sem):
    pltpu.make_async_copy(hbm_ref, vmem_ref, sem).wait()


def _scale_block(rows_v, wb, dstb, b, dst_v, lo, hi):
    def grp(g, _):
        wg = wb[b, pl.ds(g * L, L)]
        dg = dstb[b, pl.ds(g * L, L)] - lo
        inb = (dg >= 0) & (dg < hi - lo)
        dst_v[pl.ds(g * L, L)] = jnp.where(inb, dg, 0)
        wg = jnp.where(inb, wg, 0.0)
        for el in range(L):
            bw = wg.at[jnp.full((L,), el, jnp.int32)].get(
                mode="promise_in_bounds")
            for j in range(H // L):
                rows_v[g * L + el, pl.ds(j * L, L)] = (
                    rows_v[g * L + el, pl.ds(j * L, L)] * bw)
        return 0
    lax.fori_loop(0, EB // L, grp, 0)


def _sweep(h_hbm, src_hbm, dst_hbm, w_hbm, acc_sh, bufs, base_blk, nseg,
           lo, hi):
    srcb, dstb, wb, rows_v, idx_v, dst_v, sem = bufs

    for seg in range(nseg):
        r0 = pl.multiple_of(base_blk + seg * SEG, 8)
        pltpu.sync_copy(src_hbm.at[pl.ds(r0, SEG)], srcb)
        pltpu.sync_copy(dst_hbm.at[pl.ds(r0, SEG)], dstb)
        pltpu.sync_copy(w_hbm.at[pl.ds(r0, SEG)], wb)

        def body(b, _):
            for g in range(EB // L):
                idx_v[pl.ds(g * L, L)] = srcb[b, pl.ds(g * L, L)]
            pltpu.async_copy(h_hbm.at[idx_v], rows_v, sem).wait()
            _scale_block(rows_v, wb, dstb, b, dst_v, lo, hi)
            pltpu.sync_copy(rows_v, acc_sh.at[dst_v], add=True)
            return 0

        lax.fori_loop(0, SEG, body, 0)


def _zero_acc(acc_sh, rows_v, s, stripe):
    n = stripe // EB

    def zr(r, _):
        pltpu.sync_copy(rows_v, acc_sh.at[pl.ds(s * stripe + r * EB, EB)])
        return 0
    lax.fori_loop(0, n, zr, 0)
    rem = stripe - n * EB
    if rem:
        pltpu.sync_copy(rows_v.at[pl.ds(0, rem)],
                        acc_sh.at[pl.ds(s * stripe + n * EB, rem)])


def _fc_sc_kernel(h_hbm, src_hbm, dst_hbm, w_hbm, out_hbm,
                  srcb, dstb, wb, rows_v, idx_v, dst_v, sem, acc_sh):
    c = lax.axis_index("c")
    s = lax.axis_index("s")
    wid = s * 2 + c
    stripe = NCP // 16
    nseg = EP // (32 * SEG * EB)

    _zero_rows(rows_v, EB)
    _zero_acc(acc_sh, rows_v, s, stripe)
    plsc.subcore_barrier()

    bufs = (srcb, dstb, wb, rows_v, idx_v, dst_v, sem)
    _sweep(h_hbm, src_hbm, dst_hbm, w_hbm, acc_sh, bufs,
           wid * nseg * SEG, nseg, 0, NCP)
    plsc.subcore_barrier()

    pltpu.sync_copy(acc_sh.at[pl.ds(s * stripe, stripe)],
                    out_hbm.at[c, pl.ds(s * stripe, stripe)])


def _sc_scratch(table_rows):
    return [
        pltpu.VMEM((SEG, H), jnp.int32),
        pltpu.VMEM((SEG, H), jnp.int32),
        pltpu.VMEM((SEG, H), jnp.float32),
        pltpu.VMEM((EB, H), jnp.float32),
        pltpu.VMEM((EB,), jnp.int32),
        pltpu.VMEM((EB,), jnp.int32),
        pltpu.SemaphoreType.DMA,
        pltpu.VMEM_SHARED((table_rows, H), jnp.float32),
    ]


def _agg_fc(h, src2d, dst2d, w2d):
    mesh = plsc.VectorSubcoreMesh(core_axis_name="c", subcore_axis_name="s")
    return pl.kernel(
        _fc_sc_kernel,
        out_type=jax.ShapeDtypeStruct((2, NCP, H), jnp.float32),
        mesh=mesh,
        scratch_types=_sc_scratch(NCP),
    )(h, src2d, dst2d, w2d)


def _cf_sc_kernel(h_hbm, src_hbm, dst_hbm, w_hbm, out_hbm,
                  srcb, dstb, wb, rows_v, idx_v, dst_v, sem, acc_sh):
    c = lax.axis_index("c")
    s = lax.axis_index("s")
    stripe = CH // 16
    nseg = EP // (16 * SEG * EB)

    bufs = (srcb, dstb, wb, rows_v, idx_v, dst_v, sem)
    for chunk in range(2):
        lo = (c * 2 + chunk) * CH
        _zero_rows(rows_v, EB)
        _zero_acc(acc_sh, rows_v, s, stripe)
        plsc.subcore_barrier()

        _sweep(h_hbm, src_hbm, dst_hbm, w_hbm, acc_sh, bufs,
               s * nseg * SEG, nseg, lo, lo + CH)
        plsc.subcore_barrier()

        def dump(r, _):
            pltpu.sync_copy(acc_sh.at[pl.ds(s * stripe + r * 16, 16)],
                            out_hbm.at[pl.ds(lo + s * stripe + r * 16, 16)])
            return 0
        lax.fori_loop(0, stripe // 16, dump, 0)
        plsc.subcore_barrier()


def _agg_cf(h, src2d, dst2d, w2d):
    mesh = plsc.VectorSubcoreMesh(core_axis_name="c", subcore_axis_name="s")
    return pl.kernel(
        _cf_sc_kernel,
        out_type=jax.ShapeDtypeStruct((NFP, H), jnp.float32),
        mesh=mesh,
        scratch_types=_sc_scratch(CH),
    )(h, src2d, dst2d, w2d)


def kernel(x_fact, x_company, src_fc, dst_fc, src_cf, dst_cf, ea_fc, ea_cf,
           W_mix, b_mix,
           W_enc_f, b_enc_f, g_enc_f, be_enc_f,
           W_enc_c, b_enc_c, g_enc_c, be_enc_c,
           Wrel_fc_0, brel_fc_0, Wroot_fc_0,
           Wrel_cf_0, brel_cf_0, Wroot_cf_0,
           g0_f, b0_f, g0_c, b0_c,
           Wrel_fc_1, brel_fc_1, Wroot_fc_1,
           Wrel_cf_1, brel_cf_1, Wroot_cf_1,
           g1_f, b1_f, g1_c, b1_c,
           W_gate, b_gate, W_cls, b_cls):
    xf = jnp.pad(x_fact, ((0, NFP - NF), (0, 0)))
    xc = jnp.pad(x_company, ((0, NCP - NC), (0, 0)))
    pe = EP - E
    src_fc_p = jnp.pad(src_fc.astype(jnp.int32), (0, pe)).reshape(_GATE_R, H)
    dst_fc_p = jnp.pad(dst_fc.astype(jnp.int32), (0, pe)).reshape(_GATE_R, H)
    src_cf_p = jnp.pad(src_cf.astype(jnp.int32), (0, pe)).reshape(_GATE_R, H)
    dst_cf_p = jnp.pad(dst_cf.astype(jnp.int32), (0, pe)).reshape(_GATE_R, H)
    ea_fc_p = jnp.pad(ea_fc[:, 0], (0, pe)).reshape(_GATE_R, H)
    ea_cf_p = jnp.pad(ea_cf[:, 0], (0, pe)).reshape(_GATE_R, H)

    w_fc, w_cf = _gates(ea_fc_p, ea_cf_p, W_mix, b_mix)

    hf = _encode(xf, W_enc_f, b_enc_f, g_enc_f, be_enc_f, NFP)
    hc = _encode(xc, W_enc_c, b_enc_c, g_enc_c, be_enc_c, NCP)

    aggc_p = _agg_fc(hf, src_fc_p, dst_fc_p, w_fc)
    aggf = _agg_cf(hc, src_cf_p, dst_cf_p, w_cf)
    hf, _ = _update_f(aggf, hf, Wrel_cf_0, brel_cf_0, Wroot_cf_0, g0_f, b0_f)
    hc, _ = _update_c(aggc_p, hc, Wrel_fc_0, brel_fc_0, Wroot_fc_0, g0_c, b0_c)

    aggc_p = _agg_fc(hf, src_fc_p, dst_fc_p, w_fc)
    aggf = _agg_cf(hc, src_cf_p, dst_cf_p, w_cf)
    hf, sum_f = _update_f(aggf, hf, Wrel_cf_1, brel_cf_1, Wroot_cf_1,
                          g1_f, b1_f)
    hc, sum_c = _update_c(aggc_p, hc, Wrel_fc_1, brel_fc_1, Wroot_fc_1,
                          g1_c, b1_c)

    return _readout(sum_f, sum_c, W_gate, b_gate, W_cls, b_cls)


# speedup vs baseline: 1.8479x; 1.8479x over previous
import functools

import jax
import jax.numpy as jnp
from jax import lax
from jax.experimental import pallas as pl
from jax.experimental.pallas import tpu as pltpu
from jax.experimental.pallas import tpu_sc as plsc

H = 128
NF = 50000
NC = 5000
E = 500000

NFP = 50176
NCP = 5120
EP = 503808
EB = 128
NBW = 123
NBS = 246
CH = 12544
L = 16

_GATE_R = EP // H


def _gate_body(ea_fc_ref, ea_cf_ref, wm_ref, bm_ref, wfc_ref, wcf_ref):
    wm = wm_ref[0, 0]
    bm = bm_ref[0, 0]
    r = lax.broadcasted_iota(jnp.int32, (_GATE_R, H), 0)
    c = lax.broadcasted_iota(jnp.int32, (_GATE_R, H), 1)
    valid = (r * H + c) < E
    for ea_ref, w_ref in ((ea_fc_ref, wfc_ref), (ea_cf_ref, wcf_ref)):
        w = jax.nn.sigmoid(ea_ref[...] * wm + bm)
        w_ref[...] = jnp.where(valid, w, 0.0)


def _gates(ea_fc, ea_cf, W_mix, b_mix):
    return pl.pallas_call(
        _gate_body,
        out_shape=(
            jax.ShapeDtypeStruct((_GATE_R, H), jnp.float32),
            jax.ShapeDtypeStruct((_GATE_R, H), jnp.float32),
        ),
        in_specs=[
            pl.BlockSpec((_GATE_R, H), lambda: (0, 0)),
            pl.BlockSpec((_GATE_R, H), lambda: (0, 0)),
            pl.BlockSpec(memory_space=pltpu.SMEM),
            pl.BlockSpec(memory_space=pltpu.SMEM),
        ],
        out_specs=(
            pl.BlockSpec((_GATE_R, H), lambda: (0, 0)),
            pl.BlockSpec((_GATE_R, H), lambda: (0, 0)),
        ),
    )(ea_fc, ea_cf, W_mix, b_mix.reshape(1, 1))


def _ln_rows(x, g, b):
    m = jnp.mean(x, axis=-1, keepdims=True)
    v = jnp.mean(jnp.square(x - m), axis=-1, keepdims=True)
    return g * (x - m) / jnp.sqrt(v + 1e-5) + b


def _enc_body(x_ref, W_ref, b_ref, g_ref, be_ref, o_ref):
    x = jnp.dot(x_ref[...], W_ref[...].T, preferred_element_type=jnp.float32)
    x = jax.nn.relu(x + b_ref[...])
    o_ref[...] = _ln_rows(x, g_ref[...], be_ref[...])


def _encode(x, W, b, g, be, rows, blk=512):
    return pl.pallas_call(
        _enc_body,
        grid=(rows // blk,),
        out_shape=jax.ShapeDtypeStruct((rows, H), jnp.float32),
        in_specs=[
            pl.BlockSpec((blk, H), lambda i: (i, 0)),
            pl.BlockSpec((H, H), lambda i: (0, 0)),
            pl.BlockSpec((1, H), lambda i: (0, 0)),
            pl.BlockSpec((1, H), lambda i: (0, 0)),
            pl.BlockSpec((1, H), lambda i: (0, 0)),
        ],
        out_specs=pl.BlockSpec((blk, H), lambda i: (i, 0)),
    )(x, W, b.reshape(1, H), g.reshape(1, H), be.reshape(1, H))


def _upd_f_body(agg_ref, h_ref, Wrel_ref, brel_ref, Wroot_ref, g_ref, b_ref,
                o_ref, sum_ref, *, nvalid, blk):
    i = pl.program_id(0)
    x = jnp.dot(agg_ref[...], Wrel_ref[...].T, preferred_element_type=jnp.float32)
    x = x + brel_ref[...]
    x = x + jnp.dot(h_ref[...], Wroot_ref[...].T, preferred_element_type=jnp.float32)
    x = jax.nn.relu(x)
    out = _ln_rows(x, g_ref[...], b_ref[...])
    o_ref[...] = out
    rowid = i * blk + lax.broadcasted_iota(jnp.int32, (blk, 1), 0)
    masked = jnp.where(rowid < nvalid, out, 0.0)

    @pl.when(i == 0)
    def _():
        sum_ref[...] = jnp.zeros_like(sum_ref)

    sum_ref[...] += jnp.sum(masked, axis=0, keepdims=True)


def _upd_c_body(aggp_ref, h_ref, Wrel_ref, brel_ref, Wroot_ref, g_ref, b_ref,
                o_ref, sum_ref, *, nvalid, blk):
    i = pl.program_id(0)
    agg = aggp_ref[0] + aggp_ref[1]
    x = jnp.dot(agg, Wrel_ref[...].T, preferred_element_type=jnp.float32)
    x = x + brel_ref[...]
    x = x + jnp.dot(h_ref[...], Wroot_ref[...].T, preferred_element_type=jnp.float32)
    x = jax.nn.relu(x)
    out = _ln_rows(x, g_ref[...], b_ref[...])
    o_ref[...] = out
    rowid = i * blk + lax.broadcasted_iota(jnp.int32, (blk, 1), 0)
    masked = jnp.where(rowid < nvalid, out, 0.0)

    @pl.when(i == 0)
    def _():
        sum_ref[...] = jnp.zeros_like(sum_ref)

    sum_ref[...] += jnp.sum(masked, axis=0, keepdims=True)


def _update_f(agg, h, Wrel, brel, Wroot, g, b, blk=512):
    return pl.pallas_call(
        functools.partial(_upd_f_body, nvalid=NF, blk=blk),
        grid=(NFP // blk,),
        out_shape=(
            jax.ShapeDtypeStruct((NFP, H), jnp.float32),
            jax.ShapeDtypeStruct((1, H), jnp.float32),
        ),
        in_specs=[
            pl.BlockSpec((blk, H), lambda i: (i, 0)),
            pl.BlockSpec((blk, H), lambda i: (i, 0)),
            pl.BlockSpec((H, H), lambda i: (0, 0)),
            pl.BlockSpec((1, H), lambda i: (0, 0)),
            pl.BlockSpec((H, H), lambda i: (0, 0)),
            pl.BlockSpec((1, H), lambda i: (0, 0)),
            pl.BlockSpec((1, H), lambda i: (0, 0)),
        ],
        out_specs=(
            pl.BlockSpec((blk, H), lambda i: (i, 0)),
            pl.BlockSpec((1, H), lambda i: (0, 0)),
        ),
    )(agg, h, Wrel, brel.reshape(1, H), Wroot, g.reshape(1, H),
      b.reshape(1, H))


def _update_c(aggp, h, Wrel, brel, Wroot, g, b, blk=512):
    return pl.pallas_call(
        functools.partial(_upd_c_body, nvalid=NC, blk=blk),
        grid=(NCP // blk,),
        out_shape=(
            jax.ShapeDtypeStruct((NCP, H), jnp.float32),
            jax.ShapeDtypeStruct((1, H), jnp.float32),
        ),
        in_specs=[
            pl.BlockSpec((2, blk, H), lambda i: (0, i, 0)),
            pl.BlockSpec((blk, H), lambda i: (i, 0)),
            pl.BlockSpec((H, H), lambda i: (0, 0)),
            pl.BlockSpec((1, H), lambda i: (0, 0)),
            pl.BlockSpec((H, H), lambda i: (0, 0)),
            pl.BlockSpec((1, H), lambda i: (0, 0)),
            pl.BlockSpec((1, H), lambda i: (0, 0)),
        ],
        out_specs=(
            pl.BlockSpec((blk, H), lambda i: (i, 0)),
            pl.BlockSpec((1, H), lambda i: (0, 0)),
        ),
    )(aggp, h, Wrel, brel.reshape(1, H), Wroot, g.reshape(1, H),
      b.reshape(1, H))


def _readout_body(sf_ref, sc_ref, Wg_ref, bg_ref, Wc_ref, bc_ref, o_ref):
    fp = sf_ref[...] / NF
    cp = sc_ref[...] / NC
    z = (jnp.sum(fp * Wg_ref[0:1, 0:H]) + jnp.sum(cp * Wg_ref[0:1, H:2 * H])
         + bg_ref[0, 0])
    alpha = jax.nn.sigmoid(z)
    pooled = alpha * fp + (1.0 - alpha) * cp
    o_ref[...] = (jnp.sum(pooled * Wc_ref[...])
                  + bc_ref[0, 0]).reshape(1, 1)


def _readout(sum_f, sum_c, W_gate, b_gate, W_cls, b_cls):
    return pl.pallas_call(
        _readout_body,
        out_shape=jax.ShapeDtypeStruct((1, 1), jnp.float32),
        in_specs=[
            pl.BlockSpec((1, H), lambda: (0, 0)),
            pl.BlockSpec((1, H), lambda: (0, 0)),
            pl.BlockSpec((1, 2 * H), lambda: (0, 0)),
            pl.BlockSpec(memory_space=pltpu.SMEM),
            pl.BlockSpec((1, H), lambda: (0, 0)),
            pl.BlockSpec(memory_space=pltpu.SMEM),
        ],
        out_specs=pl.BlockSpec((1, 1), lambda: (0, 0)),
    )(sum_f, sum_c, W_gate, b_gate.reshape(1, 1), W_cls, b_cls.reshape(1, 1))


def _zero_rows(rows_v, n):
    def zrow(e, _):
        for j in range(H // L):
            rows_v[e, pl.ds(j * L, L)] = jnp.zeros((L,), jnp.float32)
        return 0
    lax.fori_loop(0, n, zrow, 0)


def _scale_rows(rows_v, w_v):
    def grp(g, _):
        wg = w_v[pl.ds(g * L, L)]
        for el in range(L):
            bw = wg.at[jnp.full((L,), el, jnp.int32)].get(
                mode="promise_in_bounds")
            for j in range(H // L):
                rows_v[g * L + el, pl.ds(j * L, L)] = (
                    rows_v[g * L + el, pl.ds(j * L, L)] * bw)
        return 0
    lax.fori_loop(0, EB // L, grp, 0)


def _fc_sc_kernel(h_hbm, src_hbm, dst_hbm, w_hbm, out_hbm,
                  idx_v, dst_v, w_v, rows_v, acc_sh, sem):
    c = lax.axis_index("c")
    s = lax.axis_index("s")
    wid = s * 2 + c
    stripe = NCP // 16

    _zero_rows(rows_v, 64)
    for r in range(0, stripe, 64):
        pltpu.sync_copy(rows_v.at[pl.ds(0, 64)],
                        acc_sh.at[pl.ds(s * stripe + r, 64)])
    plsc.subcore_barrier()

    def blk(b, _):
        base = (wid * NBW + b) * EB
        pltpu.sync_copy(src_hbm.at[pl.ds(base, EB)], idx_v)
        pltpu.sync_copy(dst_hbm.at[pl.ds(base, EB)], dst_v)
        pltpu.sync_copy(w_hbm.at[pl.ds(base, EB)], w_v)
        pltpu.async_copy(h_hbm.at[idx_v], rows_v, sem).wait()
        _scale_rows(rows_v, w_v)
        pltpu.sync_copy(rows_v, acc_sh.at[dst_v], add=True)
        return 0

    lax.fori_loop(0, NBW, blk, 0)
    plsc.subcore_barrier()

    for r in range(0, stripe, 64):
        pltpu.sync_copy(acc_sh.at[pl.ds(s * stripe + r, 64)],
                        out_hbm.at[c, pl.ds(s * stripe + r, 64)])


def _agg_fc(h, src, dst, w):
    mesh = plsc.VectorSubcoreMesh(core_axis_name="c", subcore_axis_name="s")
    return pl.kernel(
        _fc_sc_kernel,
        out_type=jax.ShapeDtypeStruct((2, NCP, H), jnp.float32),
        mesh=mesh,
        scratch_types=[
            pltpu.VMEM((EB,), jnp.int32),
            pltpu.VMEM((EB,), jnp.int32),
            pltpu.VMEM((EB,), jnp.float32),
            pltpu.VMEM((EB, H), jnp.float32),
            pltpu.VMEM_SHARED((NCP, H), jnp.float32),
            pltpu.SemaphoreType.DMA,
        ],
    )(h, src, dst, w)


def _cf_sc_kernel(h_hbm, src_hbm, dst_hbm, w_hbm, out_hbm,
                  idx_v, dst_v, w_v, rows_v, zero_v, acc_sh, sem):
    c = lax.axis_index("c")
    s = lax.axis_index("s")
    stripe = CH // 16

    _zero_rows(zero_v, 16)

    for chunk in range(2):
        lo = (c * 2 + chunk) * CH

        def zr(r, _):
            pltpu.sync_copy(zero_v.at[pl.ds(0, 16)],
                            acc_sh.at[pl.ds(s * stripe + r * 16, 16)])
            return 0
        lax.fori_loop(0, stripe // 16, zr, 0)
        plsc.subcore_barrier()

        def blk(b, _):
            base = (s * NBS + b) * EB
            pltpu.sync_copy(src_hbm.at[pl.ds(base, EB)], idx_v)
            pltpu.sync_copy(dst_hbm.at[pl.ds(base, EB)], dst_v)
            pltpu.sync_copy(w_hbm.at[pl.ds(base, EB)], w_v)
            pltpu.async_copy(h_hbm.at[idx_v], rows_v, sem).wait()

            def grp(g, _):
                dg = dst_v[pl.ds(g * L, L)] - lo
                inb = (dg >= 0) & (dg < CH)
                dst_v[pl.ds(g * L, L)] = jnp.where(inb, dg, 0)
                w_v[pl.ds(g * L, L)] = jnp.where(
                    inb, w_v[pl.ds(g * L, L)], 0.0)
                return 0
            lax.fori_loop(0, EB // L, grp, 0)

            _scale_rows(rows_v, w_v)
            pltpu.sync_copy(rows_v, acc_sh.at[dst_v], add=True)
            return 0

        lax.fori_loop(0, NBS, blk, 0)
        plsc.subcore_barrier()

        def dump(r, _):
            pltpu.sync_copy(acc_sh.at[pl.ds(s * stripe + r * 16, 16)],
                            out_hbm.at[pl.ds(lo + s * stripe + r * 16, 16)])
            return 0
        lax.fori_loop(0, stripe // 16, dump, 0)
        plsc.subcore_barrier()


def _agg_cf(h, src, dst, w):
    mesh = plsc.VectorSubcoreMesh(core_axis_name="c", subcore_axis_name="s")
    return pl.kernel(
        _cf_sc_kernel,
        out_type=jax.ShapeDtypeStruct((NFP, H), jnp.float32),
        mesh=mesh,
        scratch_types=[
            pltpu.VMEM((EB,), jnp.int32),
            pltpu.VMEM((EB,), jnp.int32),
            pltpu.VMEM((EB,), jnp.float32),
            pltpu.VMEM((EB, H), jnp.float32),
            pltpu.VMEM((16, H), jnp.float32),
            pltpu.VMEM_SHARED((CH, H), jnp.float32),
            pltpu.SemaphoreType.DMA,
        ],
    )(h, src, dst, w)


def kernel(x_fact, x_company, src_fc, dst_fc, src_cf, dst_cf, ea_fc, ea_cf,
           W_mix, b_mix,
           W_enc_f, b_enc_f, g_enc_f, be_enc_f,
           W_enc_c, b_enc_c, g_enc_c, be_enc_c,
           Wrel_fc_0, brel_fc_0, Wroot_fc_0,
           Wrel_cf_0, brel_cf_0, Wroot_cf_0,
           g0_f, b0_f, g0_c, b0_c,
           Wrel_fc_1, brel_fc_1, Wroot_fc_1,
           Wrel_cf_1, brel_cf_1, Wroot_cf_1,
           g1_f, b1_f, g1_c, b1_c,
           W_gate, b_gate, W_cls, b_cls):
    xf = jnp.pad(x_fact, ((0, NFP - NF), (0, 0)))
    xc = jnp.pad(x_company, ((0, NCP - NC), (0, 0)))
    pe = EP - E
    src_fc_p = jnp.pad(src_fc.astype(jnp.int32), (0, pe))
    dst_fc_p = jnp.pad(dst_fc.astype(jnp.int32), (0, pe))
    src_cf_p = jnp.pad(src_cf.astype(jnp.int32), (0, pe))
    dst_cf_p = jnp.pad(dst_cf.astype(jnp.int32), (0, pe))
    ea_fc_p = jnp.pad(ea_fc[:, 0], (0, pe)).reshape(_GATE_R, H)
    ea_cf_p = jnp.pad(ea_cf[:, 0], (0, pe)).reshape(_GATE_R, H)

    w_fc2, w_cf2 = _gates(ea_fc_p, ea_cf_p, W_mix, b_mix)
    w_fc = w_fc2.reshape(EP)
    w_cf = w_cf2.reshape(EP)

    hf = _encode(xf, W_enc_f, b_enc_f, g_enc_f, be_enc_f, NFP)
    hc = _encode(xc, W_enc_c, b_enc_c, g_enc_c, be_enc_c, NCP)

    aggc_p = _agg_fc(hf, src_fc_p, dst_fc_p, w_fc)
    aggf = _agg_cf(hc, src_cf_p, dst_cf_p, w_cf)
    hf, _ = _update_f(aggf, hf, Wrel_cf_0, brel_cf_0, Wroot_cf_0, g0_f, b0_f)
    hc, _ = _update_c(aggc_p, hc, Wrel_fc_0, brel_fc_0, Wroot_fc_0, g0_c, b0_c)

    aggc_p = _agg_fc(hf, src_fc_p, dst_fc_p, w_fc)
    aggf = _agg_cf(hc, src_cf_p, dst_cf_p, w_cf)
    hf, sum_f = _update_f(aggf, hf, Wrel_cf_1, brel_cf_1, Wroot_cf_1,
                          g1_f, b1_f)
    hc, sum_c = _update_c(aggc_p, hc, Wrel_fc_1, brel_fc_1, Wroot_fc_1,
                          g1_c, b1_c)

    return _readout(sum_f, sum_c, W_gate, b_gate, W_cls, b_cls)
